# baseline (speedup 1.0000x reference)
import jax
import jax.numpy as jnp
import numpy as np
from jax.experimental import pallas as pl
from jax.experimental.pallas import tpu as pltpu

N = 50000
E = 800000
F = 64
NBASIS = 8
NLAYERS = 4
RMAX = 5.0
AVG_NEIGH = 15.0


def _sum_kernel(x_ref, o_ref):
    o_ref[0, 0] = jnp.sum(x_ref[...])


def kernel(pos, z, edge_index, type_embed, rW1, rb1, rW2, rb2, Wself, Wmsg, readW1, readb1, readW2, readb2, shifts, scales):
    src = edge_index[0]
    dst = edge_index[1]
    rvec = pos[dst] - pos[src]
    r = jnp.sqrt(jnp.sum(rvec * rvec, axis=-1) + 1e-12)
    nfreq = jnp.arange(1, NBASIS + 1, dtype=jnp.float32)
    rb = jnp.sqrt(2.0 / RMAX) * jnp.sin(nfreq[None, :] * jnp.pi * r[:, None] / RMAX) / (r[:, None] + 1e-9)
    x = r / RMAX
    p = 6.0
    cut = 1.0 - ((p + 1.0) * (p + 2.0) / 2.0) * x ** p + p * (p + 2.0) * x ** (p + 1.0) - (p * (p + 1.0) / 2.0) * x ** (p + 2.0)
    cut = jnp.where(x < 1.0, cut, 0.0)
    edge_feat = rb * cut[:, None]
    h = type_embed[z]
    for l in range(NLAYERS):
        w = jax.nn.silu(edge_feat @ rW1[l] + rb1[l]) @ rW2[l] + rb2[l]
        msg = w * h[src]
        agg = jnp.zeros((N, F), dtype=h.dtype).at[dst].add(msg) / AVG_NEIGH
        h = jax.nn.silu(h @ Wself[l] + agg @ Wmsg[l])
    e = jax.nn.silu(h @ readW1 + readb1) @ readW2 + readb2
    e = e[:, 0] * scales[z] + shifts[z]
    npad = ((N + 1023) // 1024) * 1024
    e_pad = jnp.pad(e, (0, npad - N)).reshape(npad // 128, 128)
    total = pl.pallas_call(
        _sum_kernel,
        out_shape=jax.ShapeDtypeStruct((1, 1), jnp.float32),
        out_specs=pl.BlockSpec(memory_space=pltpu.SMEM),
    )(e_pad)
    return total.reshape(1)


# speedup vs baseline: 1.3718x; 1.3718x over previous
import functools

import jax
import jax.numpy as jnp
import numpy as np
from jax import lax
from jax.experimental import pallas as pl
from jax.experimental.pallas import tpu as pltpu
from jax.experimental.pallas import tpu_sc as plsc

N = 50000
E = 800000
F = 64
NTYPES = 32
NBASIS = 8
NLAYERS = 4
RMAX = 5.0
AVG_NEIGH = 15.0
HID = 64

NC = 2
NS = 16
NW = NC * NS
CH = 128
GRP = 8
TPW = 200
NGRP = TPW // GRP
GROWS = GRP * CH
EPW = TPW * CH
EPAD = NW * EPW
NROWS_IDX = EPAD // CH

SGRP = 2
SGROWS = SGRP * CH
SNGRP = TPW // SGRP

NH = N // 2
NHPAD = 25088
STRIPE = NHPAD // NS
DUMP = NH + 8

NPAD = 51200
BN = 2048
NBLK_N = NPAD // BN
BEF = 4096
BE = 2048


def _silu(x):
    return x * jax.nn.sigmoid(x)


def _make_sc_gather(d, interpret=False):
    mesh = plsc.VectorSubcoreMesh(core_axis_name="c", subcore_axis_name="s",
                                  num_cores=NC, num_subcores=NS)

    def body(table_hbm, idx_hbm, out_hbm, idx_v, rows_v, sem):
        c = lax.axis_index("c")
        s = lax.axis_index("s")
        wid = s * NC + c
        pltpu.sync_copy(idx_hbm.at[pl.ds(wid * TPW, TPW)], idx_v)

        def grp(g):
            cps = []
            for j in range(GRP):
                cps.append(pltpu.async_copy(
                    table_hbm.at[idx_v.at[g * GRP + j]],
                    rows_v.at[pl.ds(j * CH, CH)], sem))
            for cp in cps:
                cp.wait()
            pltpu.sync_copy(rows_v,
                            out_hbm.at[pl.ds(wid * EPW + g * GROWS, GROWS)])

        pl.loop(0, NGRP)(grp)

    return pl.kernel(
        body,
        out_type=jax.ShapeDtypeStruct((EPAD, d), jnp.float32),
        mesh=mesh,
        scratch_types=[
            pltpu.VMEM((TPW, CH), jnp.int32),
            pltpu.VMEM((GROWS, d), jnp.float32),
            pltpu.SemaphoreType.DMA,
        ],
        compiler_params=pltpu.CompilerParams(use_tc_tiling_on_sc=False),
        interpret=interpret,
    )


def _make_sc_scatter(interpret=False):
    mesh = plsc.VectorSubcoreMesh(core_axis_name="c", subcore_axis_name="s",
                                  num_cores=NC, num_subcores=NS)

    def body(msg_hbm, idx_hbm, out_hbm, idx_v, msg_v, agg_sp):
        c = lax.axis_index("c")
        s = lax.axis_index("s")
        wid = s * NC + c

        z16 = jnp.zeros((16,), jnp.float32)

        def zrow(r):
            for q in range(F // 16):
                msg_v[r, pl.ds(q * 16, 16)] = z16

        pl.loop(0, SGROWS)(zrow)

        def zcp(k):
            pltpu.sync_copy(msg_v,
                            agg_sp.at[pl.ds(s * STRIPE + k * SGROWS, SGROWS)])

        pl.loop(0, STRIPE // SGROWS)(zcp)
        pltpu.sync_copy(
            msg_v.at[pl.ds(0, STRIPE % SGROWS)],
            agg_sp.at[pl.ds(s * STRIPE + (STRIPE // SGROWS) * SGROWS,
                            STRIPE % SGROWS)])
        plsc.subcore_barrier()

        def grp(g):
            pltpu.sync_copy(idx_hbm.at[pl.ds(wid * TPW + g * SGRP, SGRP)],
                            idx_v)
            pltpu.sync_copy(msg_hbm.at[pl.ds(wid * EPW + g * SGROWS, SGROWS)],
                            msg_v)
            for j in range(SGRP):
                pltpu.sync_copy(msg_v.at[pl.ds(j * CH, CH)],
                                agg_sp.at[idx_v.at[j]], add=True)

        pl.loop(0, SNGRP)(grp)
        plsc.subcore_barrier()

        pltpu.sync_copy(agg_sp.at[pl.ds(s * STRIPE, STRIPE)],
                        out_hbm.at[c, pl.ds(s * STRIPE, STRIPE)])

    return pl.kernel(
        body,
        out_type=jax.ShapeDtypeStruct((NC, NHPAD, F), jnp.float32),
        mesh=mesh,
        scratch_types=[
            pltpu.VMEM((SGRP, CH), jnp.int32),
            pltpu.VMEM((SGROWS, F), jnp.float32),
            pltpu.VMEM_SHARED((NHPAD, F), jnp.float32),
        ],
        compiler_params=pltpu.CompilerParams(use_tc_tiling_on_sc=False),
        interpret=interpret,
    )


def _embed_body(z_ref, te_ref, o_ref):
    z = z_ref[0, 0, :]
    oh = (z[:, None] == lax.broadcasted_iota(jnp.int32, (BN, NTYPES), 1))
    o_ref[...] = jnp.dot(oh.astype(jnp.float32), te_ref[...],
                         preferred_element_type=jnp.float32)


def _tc_embed(z3, type_embed, interpret=False):
    return pl.pallas_call(
        _embed_body,
        grid=(NBLK_N,),
        in_specs=[
            pl.BlockSpec((1, 1, BN), lambda i: (i, 0, 0)),
            pl.BlockSpec((NTYPES, F), lambda i: (0, 0)),
        ],
        out_specs=pl.BlockSpec((BN, F), lambda i: (i, 0)),
        out_shape=jax.ShapeDtypeStruct((NPAD, F), jnp.float32),
        interpret=interpret,
    )(z3, type_embed)


def _edgefeat_body(ps_ref, pd_ref, o_ref):
    d = pd_ref[...] - ps_ref[...]
    colmask = (lax.broadcasted_iota(jnp.int32, (BEF, 16), 1) < 3)
    d2 = jnp.where(colmask, d * d, 0.0)
    r2 = jnp.sum(d2, axis=1, keepdims=True)
    r = jnp.sqrt(r2 + 1e-12)
    x = r / RMAX
    x2 = x * x
    x3 = x2 * x
    x6 = x3 * x3
    cut = 1.0 - x6 * (28.0 - 48.0 * x + 21.0 * x2)
    cut = jnp.where(x < 1.0, cut, 0.0)
    nf = (lax.broadcasted_iota(jnp.int32, (1, NBASIS), 1).astype(jnp.float32)
          + 1.0)
    rb = np.sqrt(2.0 / RMAX) * jnp.sin(nf * (np.pi / RMAX) * r) / (r + 1e-9)
    o_ref[...] = rb * cut


def _tc_edgefeat(ps, pd, interpret=False):
    return pl.pallas_call(
        _edgefeat_body,
        grid=(EPAD // BEF,),
        in_specs=[
            pl.BlockSpec((BEF, 16), lambda i: (i, 0)),
            pl.BlockSpec((BEF, 16), lambda i: (i, 0)),
        ],
        out_specs=pl.BlockSpec((BEF, NBASIS), lambda i: (i, 0)),
        out_shape=jax.ShapeDtypeStruct((EPAD, NBASIS), jnp.float32),
        interpret=interpret,
    )(ps, pd)


def _msg_body(ef_ref, hs_ref, w1_ref, b1_ref, w2_ref, b2_ref, o_ref):
    a = _silu(jnp.dot(ef_ref[...], w1_ref[...],
                      preferred_element_type=jnp.float32) + b1_ref[...])
    w = jnp.dot(a, w2_ref[...], preferred_element_type=jnp.float32) + b2_ref[...]
    o_ref[...] = w * hs_ref[...]


def _tc_msg(ef, hs, w1, b1, w2, b2, interpret=False):
    return pl.pallas_call(
        _msg_body,
        grid=(EPAD // BE,),
        in_specs=[
            pl.BlockSpec((BE, NBASIS), lambda i: (i, 0)),
            pl.BlockSpec((BE, F), lambda i: (i, 0)),
            pl.BlockSpec((NBASIS, HID), lambda i: (0, 0)),
            pl.BlockSpec((1, HID), lambda i: (0, 0)),
            pl.BlockSpec((HID, F), lambda i: (0, 0)),
            pl.BlockSpec((1, F), lambda i: (0, 0)),
        ],
        out_specs=pl.BlockSpec((BE, F), lambda i: (i, 0)),
        out_shape=jax.ShapeDtypeStruct((EPAD, F), jnp.float32),
        interpret=interpret,
    )(ef, hs, w1, b1, w2, b2)


def _hupd_body(h_ref, agg_ref, ws_ref, wm_ref, o_ref):
    hp = jnp.dot(h_ref[...], ws_ref[...], preferred_element_type=jnp.float32)
    ap = jnp.dot(agg_ref[...] * (1.0 / AVG_NEIGH), wm_ref[...],
                 preferred_element_type=jnp.float32)
    o_ref[...] = _silu(hp + ap)


def _tc_hupd(h, agg, ws, wm, interpret=False):
    return pl.pallas_call(
        _hupd_body,
        grid=(NBLK_N,),
        in_specs=[
            pl.BlockSpec((BN, F), lambda i: (i, 0)),
            pl.BlockSpec((BN, F), lambda i: (i, 0)),
            pl.BlockSpec((F, F), lambda i: (0, 0)),
            pl.BlockSpec((F, F), lambda i: (0, 0)),
        ],
        out_specs=pl.BlockSpec((BN, F), lambda i: (i, 0)),
        out_shape=jax.ShapeDtypeStruct((NPAD, F), jnp.float32),
        interpret=interpret,
    )(h, agg, ws, wm)


def _read_body(h_ref, z_ref, w1_ref, b1_ref, w2_ref, b2_ref,
               sc_ref, sh_ref, o_ref):
    i = pl.program_id(0)
    s1 = _silu(jnp.dot(h_ref[...], w1_ref[...],
                       preferred_element_type=jnp.float32) + b1_ref[...])
    e = jnp.dot(s1, w2_ref[...], preferred_element_type=jnp.float32) + b2_ref[...]
    z = z_ref[0, 0, :]
    oh = (z[:, None] == lax.broadcasted_iota(jnp.int32, (BN, NTYPES), 1))
    ohf = oh.astype(jnp.float32)
    scv = jnp.sum(ohf * sc_ref[...], axis=1)
    shv = jnp.sum(ohf * sh_ref[...], axis=1)
    row = i * BN + lax.broadcasted_iota(jnp.int32, (BN,), 0)
    val = jnp.where(row < N, e[:, 0] * scv + shv, 0.0)

    @pl.when(i == 0)
    def _():
        o_ref[0, 0] = 0.0

    o_ref[0, 0] += jnp.sum(val)


def _tc_read(h, z3, w1, b1, w2, b2, sc, sh, interpret=False):
    return pl.pallas_call(
        _read_body,
        grid=(NBLK_N,),
        in_specs=[
            pl.BlockSpec((BN, F), lambda i: (i, 0)),
            pl.BlockSpec((1, 1, BN), lambda i: (i, 0, 0)),
            pl.BlockSpec((F, 32), lambda i: (0, 0)),
            pl.BlockSpec((1, 32), lambda i: (0, 0)),
            pl.BlockSpec((32, 1), lambda i: (0, 0)),
            pl.BlockSpec((1, 1), lambda i: (0, 0)),
            pl.BlockSpec((1, NTYPES), lambda i: (0, 0)),
            pl.BlockSpec((1, NTYPES), lambda i: (0, 0)),
        ],
        out_specs=pl.BlockSpec((1, 1), lambda i: (0, 0),
                               memory_space=pltpu.SMEM),
        out_shape=jax.ShapeDtypeStruct((1, 1), jnp.float32),
        interpret=interpret,
    )(h, z3, w1, b1, w2, b2, sc, sh)


def _run(pos, z, edge_index, type_embed, rW1, rb1, rW2, rb2, Wself, Wmsg,
         readW1, readb1, readW2, readb2, shifts, scales,
         interpret=False):
    src = edge_index[0].astype(jnp.int32)
    dst = edge_index[1].astype(jnp.int32)

    srcp = jnp.pad(src, (0, EPAD - E))
    dstp_g = jnp.pad(dst, (0, EPAD - E))
    dstp = jnp.pad(dst, (0, EPAD - E), constant_values=N)
    src2d = srcp.reshape(NROWS_IDX, CH)
    dstg2d = dstp_g.reshape(NROWS_IDX, CH)
    dstA2d = jnp.where(dstp < NH, dstp, DUMP).reshape(NROWS_IDX, CH)
    dstB2d = jnp.where((dstp >= NH) & (dstp < N), dstp - NH,
                       DUMP).reshape(NROWS_IDX, CH)

    pos16 = jnp.pad(pos, ((0, 0), (0, 13)))
    zp = jnp.pad(z.astype(jnp.int32), (0, NPAD - N))
    z3 = zp.reshape(NBLK_N, 1, BN)

    gather16 = _make_sc_gather(16, interpret)
    gather64 = _make_sc_gather(F, interpret)
    scatter = _make_sc_scatter(interpret)

    ps = gather16(pos16, src2d)
    pd = gather16(pos16, dstg2d)
    ef = _tc_edgefeat(ps, pd, interpret)

    h = _tc_embed(z3, type_embed, interpret)
    for l in range(NLAYERS):
        hs = gather64(h, src2d)
        msg = _tc_msg(ef, hs, rW1[l], rb1[l].reshape(1, HID),
                      rW2[l], rb2[l].reshape(1, F), interpret)
        pA = scatter(msg, dstA2d)
        pB = scatter(msg, dstB2d)
        agg = jnp.concatenate(
            [pA[0, :NH] + pA[1, :NH], pB[0, :NH] + pB[1, :NH]], axis=0)
        agg = jnp.pad(agg, ((0, NPAD - N), (0, 0)))
        h = _tc_hupd(h, agg, Wself[l], Wmsg[l], interpret)

    tot = _tc_read(h, z3, readW1, readb1.reshape(1, 32),
                   readW2, readb2.reshape(1, 1),
                   scales.reshape(1, NTYPES), shifts.reshape(1, NTYPES),
                   interpret)
    return tot.reshape(1)


def kernel(pos, z, edge_index, type_embed, rW1, rb1, rW2, rb2, Wself, Wmsg,
           readW1, readb1, readW2, readb2, shifts, scales):
    return _run(pos, z, edge_index, type_embed, rW1, rb1, rW2, rb2,
                Wself, Wmsg, readW1, readb1, readW2, readb2, shifts, scales)
